# Initial kernel scaffold; baseline (speedup 1.0000x reference)
import jax
import jax.numpy as jnp
from jax.experimental import pallas as pl


def kernel(x, edge_index, edge_weight, W1, att_src1, att_dst1, att_edge1, W_edge1, b1, W2, att_src2, att_dst2, att_edge2, W_edge2, b2):
    raise NotImplementedError("write your pallas kernel here")


# speedup vs baseline: 1.0235x; 1.0235x over previous
import functools

import jax
import jax.numpy as jnp
from jax.experimental import pallas as pl
from jax.experimental.pallas import tpu as pltpu

N = 10000
E = 160000
IN_CH = 256
HID = 256
OUT_CH = 256
HEADS = 8

NPAD = 10240
BLK = 256


def _mm_att_body(x_ref, w_ref, a_ref, h_ref, ab_ref):
    h = jnp.dot(x_ref[...], w_ref[...], preferred_element_type=jnp.float32)
    h_ref[...] = h
    ab_ref[...] = jnp.dot(h, a_ref[...], preferred_element_type=jnp.float32)


def _mm_att(x, w, amat):
    n, k = x.shape
    m = w.shape[1]
    j = amat.shape[1]
    grid = (n // BLK,)
    return pl.pallas_call(
        _mm_att_body,
        grid=grid,
        in_specs=[
            pl.BlockSpec((BLK, k), lambda i: (i, 0)),
            pl.BlockSpec((k, m), lambda i: (0, 0)),
            pl.BlockSpec((m, j), lambda i: (0, 0)),
        ],
        out_specs=[
            pl.BlockSpec((BLK, m), lambda i: (i, 0)),
            pl.BlockSpec((BLK, j), lambda i: (i, 0)),
        ],
        out_shape=[
            jax.ShapeDtypeStruct((n, m), jnp.float32),
            jax.ShapeDtypeStruct((n, j), jnp.float32),
        ],
    )(x, w, amat)


def _blockdiag(att, heads, hid):
    a = att.reshape(heads, hid)
    eye = jnp.eye(heads, dtype=a.dtype)
    return (a[:, :, None] * eye[:, None, :]).reshape(heads * hid, heads)


def _segment_softmax_agg(h, alpha_src, alpha_dst, cvec, edge_index, edge_weight, heads, hid):
    src = edge_index[0]
    dst = edge_index[1]
    alpha = alpha_src[src] + alpha_dst[dst] + edge_weight[:, None] * cvec[None, :]
    alpha = jax.nn.leaky_relu(alpha, negative_slope=0.2)
    amax = jax.ops.segment_max(alpha, dst, num_segments=N)
    amax = jnp.where(jnp.isfinite(amax), amax, 0.0)
    ex = jnp.exp(alpha - amax[dst])
    denom = jax.ops.segment_sum(ex, dst, num_segments=N)
    coef = ex / (denom[dst] + 1e-16)
    msg = h[src].reshape(E, heads, hid) * coef[:, :, None]
    return jax.ops.segment_sum(msg, dst, num_segments=N).reshape(N, heads * hid)


def kernel(x, edge_index, edge_weight, W1, att_src1, att_dst1, att_edge1, W_edge1, b1,
           W2, att_src2, att_dst2, att_edge2, W_edge2, b2):
    xp = jnp.pad(x, ((0, NPAD - N), (0, 0)))

    amat1 = jnp.concatenate(
        [_blockdiag(att_src1, HEADS, HID), _blockdiag(att_dst1, HEADS, HID)], axis=1)
    h1p, ab1p = _mm_att(xp, W1, amat1)
    h1 = h1p[:N]
    asrc1 = ab1p[:N, :HEADS]
    adst1 = ab1p[:N, HEADS:]
    c1 = (W_edge1.reshape(HEADS, HID) * att_edge1[0]).sum(-1)

    out1 = _segment_softmax_agg(h1, asrc1, adst1, c1, edge_index, edge_weight, HEADS, HID)
    hfeat = jax.nn.elu(out1 + b1)

    hp = jnp.pad(hfeat, ((0, NPAD - N), (0, 0)))
    amat2 = jnp.concatenate(
        [_blockdiag(att_src2, 1, OUT_CH), _blockdiag(att_dst2, 1, OUT_CH)], axis=1)
    h2p, ab2p = _mm_att(hp, W2, amat2)
    h2 = h2p[:N]
    asrc2 = ab2p[:N, :1]
    adst2 = ab2p[:N, 1:]
    c2 = (W_edge2.reshape(1, OUT_CH) * att_edge2[0]).sum(-1)

    out2 = _segment_softmax_agg(h2, asrc2, adst2, c2, edge_index, edge_weight, 1, OUT_CH)
    out2 = out2 + b2
    return jax.nn.log_softmax(out2, axis=1)


# speedup vs baseline: 7.1382x; 6.9743x over previous
import functools

import jax
import jax.numpy as jnp
from jax import lax
from jax.experimental import pallas as pl
from jax.experimental.pallas import tpu as pltpu
from jax.experimental.pallas import tpu_sc as plsc

N = 10000
E = 160000
IN_CH = 256
HID = 256
OUT_CH = 256
HEADS = 8

NPAD = 10240
BLK = 256
NC = 2
NS = 16
L = 16
CH = 256
NCHUNK = NPAD // CH
CPC = NCHUNK // NC
ACC_ROWS = 384
ZR_PER_TILE = ACC_ROWS // NS
EPT = E // NS
SEL_CAP = EPT + L


def _mm_att_body(prologue, raw_ref, den_ref, b_ref, exp_ref,
                 w_ref, amat_ref, we_ref, ae_ref,
                 h_ref, ab_ref, auxc_ref, auxm_ref):
    i = pl.program_id(0)
    nb = pl.num_programs(0)
    if prologue:
        den = den_ref[...][:, :HEADS]
        rden = jnp.where(den > 0, 1.0 / den, 0.0)
        scale = jnp.dot(rden, exp_ref[...], preferred_element_type=jnp.float32)
        hf = raw_ref[...] * scale + b_ref[...]
        x = jnp.where(hf > 0, hf, jnp.exp(jnp.minimum(hf, 0.0)) - 1.0)
    else:
        x = raw_ref[...]
    h = jnp.dot(x, w_ref[...], preferred_element_type=jnp.float32)
    h_ref[...] = h
    ab = jnp.dot(h, amat_ref[...], preferred_element_type=jnp.float32)
    ab_ref[...] = ab
    bmax = jnp.max(ab[:, :HEADS], axis=0)
    colm = jnp.broadcast_to(bmax[:, None], (HEADS, 128))
    c = jnp.sum(we_ref[...] * ae_ref[...], axis=1)

    @pl.when(i == 0)
    def _():
        auxc_ref[...] = jnp.broadcast_to(c[:, None], (HEADS, 128))
        auxm_ref[...] = colm

    @pl.when(i > 0)
    def _():
        auxm_ref[...] = jnp.maximum(auxm_ref[...], colm)

    @pl.when(i == nb - 1)
    def _():
        auxm_ref[...] = auxm_ref[...] + jnp.broadcast_to(
            jnp.maximum(c, 0.0)[:, None], (HEADS, 128))


def _mm_att(x_or_raw, den, b, expand, w, amat, we, ae, prologue):
    n = x_or_raw.shape[0]
    k = x_or_raw.shape[1]
    m = w.shape[1]
    grid = (n // BLK,)
    full = lambda shape: pl.BlockSpec(shape, lambda i: tuple(0 for _ in shape))
    return pl.pallas_call(
        functools.partial(_mm_att_body, prologue),
        grid=grid,
        in_specs=[
            pl.BlockSpec((BLK, k), lambda i: (i, 0)),
            pl.BlockSpec((BLK, 16), lambda i: (i, 0)),
            full((1, k)),
            full((HEADS, k)),
            full((k, m)),
            full((m, 16)),
            full((HEADS, 256)),
            full((HEADS, 256)),
        ],
        out_specs=[
            pl.BlockSpec((BLK, m), lambda i: (i, 0)),
            pl.BlockSpec((BLK, 16), lambda i: (i, 0)),
            full((HEADS, 128)),
            full((HEADS, 128)),
        ],
        out_shape=[
            jax.ShapeDtypeStruct((n, m), jnp.float32),
            jax.ShapeDtypeStruct((n, 16), jnp.float32),
            jax.ShapeDtypeStruct((HEADS, 128), jnp.float32),
            jax.ShapeDtypeStruct((HEADS, 128), jnp.float32),
        ],
    )(x_or_raw, den, b, expand, w, amat, we, ae)


def _final_body(raw_ref, den_ref, b_ref, out_ref):
    den0 = den_ref[...][:, 0:1]
    rden = jnp.where(den0 > 0, 1.0 / den0, 0.0)
    o = raw_ref[...] * rden + b_ref[...]
    t = o - jnp.max(o, axis=1, keepdims=True)
    lse = jnp.log(jnp.sum(jnp.exp(t), axis=1, keepdims=True))
    out_ref[...] = t - lse


def _final(raw, den, b):
    n = raw.shape[0]
    f = raw.shape[1]
    return pl.pallas_call(
        _final_body,
        grid=(n // BLK,),
        in_specs=[
            pl.BlockSpec((BLK, f), lambda i: (i, 0)),
            pl.BlockSpec((BLK, 16), lambda i: (i, 0)),
            pl.BlockSpec((1, f), lambda i: (0, 0)),
        ],
        out_specs=pl.BlockSpec((BLK, f), lambda i: (i, 0)),
        out_shape=jax.ShapeDtypeStruct((n, f), jnp.float32),
    )(raw, den, b)


def _sc_edge_body(F, H, stage, h_hbm, ab_hbm, auxc_hbm, auxm_hbm, src_hbm, dst_hbm,
                  ew_hbm, zeros_hbm, raw_hbm, den_hbm,
                  src_sl, dst_sl, ew_sl, sel, rows, astat, dstat, exden,
                  auxc_v, auxm_v, srcidx, dstidx, lidx, zden,
                  acc_msg, acc_den):
    cid = lax.axis_index("c")
    sid = lax.axis_index("s")
    iot = lax.iota(jnp.int32, L)
    zeros16 = jnp.zeros((L,), jnp.float32)

    ebase = sid * EPT
    pltpu.sync_copy(src_hbm.at[pl.ds(ebase, EPT)], src_sl)
    pltpu.sync_copy(dst_hbm.at[pl.ds(ebase, EPT)], dst_sl)
    pltpu.sync_copy(ew_hbm.at[pl.ds(ebase, EPT)], ew_sl)
    pltpu.sync_copy(auxc_hbm, auxc_v)
    pltpu.sync_copy(auxm_hbm, auxm_v)

    def zdloop(j, _):
        zden[j, pl.ds(0, L)] = zeros16
        return 0
    lax.fori_loop(0, ZR_PER_TILE, zdloop, 0)

    for h in range(H, 8):
        plsc.store_scatter(exden, [iot, jnp.full((L,), h, jnp.int32)], zeros16)
        plsc.store_scatter(exden, [iot, jnp.full((L,), h + 8, jnp.int32)], zeros16)
    if H < 8:
        plsc.store_scatter(exden, [iot, jnp.full((L,), 8, jnp.int32)], zeros16)

    def chunk_body(ci, _):
        base = pl.multiple_of((ci * NC + cid) * CH, 8)
        zr = pl.multiple_of(sid * ZR_PER_TILE, 8)
        for j in range(ZR_PER_TILE // 8):
            pltpu.sync_copy(zeros_hbm, acc_msg.at[pl.ds(zr + j * 8, 8)])
        pltpu.sync_copy(zden, acc_den.at[pl.ds(zr, ZR_PER_TILE)])
        plsc.subcore_barrier()

        def scan_body(j, cnt):
            d16 = dst_sl[pl.ds(j * L, L)]
            msk = (d16 >= base) & (d16 < base + CH)
            pref = plsc.cumsum(msk.astype(jnp.int32))
            plsc.store_scatter(sel, [cnt + pref - 1], j * L + iot, mask=msk)
            return cnt + pref[L - 1]
        if stage >= 1:
            cnt = lax.fori_loop(0, EPT // L, scan_body, jnp.int32(0))
        else:
            cnt = jnp.int32(0)

        def grp_body(g, _):
            ids = sel[pl.ds(g * L, L)]
            valid = (g * L + iot) < cnt
            ids = jnp.where(valid, ids, 0)
            s16 = plsc.load_gather(src_sl, [ids])
            d16 = plsc.load_gather(dst_sl, [ids])
            ew16 = plsc.load_gather(ew_sl, [ids])
            srcidx[...] = s16
            dstidx[...] = d16
            lidx[...] = jnp.where(valid, d16 - base, CH + 8 * sid)
            pltpu.sync_copy(h_hbm.at[srcidx], rows)
            pltpu.sync_copy(ab_hbm.at[srcidx], astat)
            pltpu.sync_copy(ab_hbm.at[dstidx], dstat)
            validf = valid.astype(jnp.float32)
            for h in range(H if stage >= 3 else 0):
                hcol = jnp.full((L,), h, jnp.int32)
                a_s = plsc.load_gather(astat, [iot, hcol])
                a_d = plsc.load_gather(dstat, [iot, hcol + 8])
                c_h = auxc_v[h, pl.ds(0, L)][0]
                mp_h = auxm_v[h, pl.ds(0, L)][0]
                al = a_s + a_d + ew16 * c_h
                al = jnp.where(al > 0, al, 0.2 * al)
                mb = a_d + mp_h
                mb = jnp.where(mb > 0, mb, 0.2 * mb)
                exh = jnp.exp(al - mb) * validf
                plsc.store_scatter(exden, [iot, hcol], exh)

            def scale_body(i, _):
                exr = exden[i, pl.ds(0, L)]
                for h in range(H):
                    e_s = exr[h]
                    for k in range(256 // L):
                        sl = pl.ds(h * 256 + k * L, L)
                        rows[i, sl] = rows[i, sl] * e_s
                return 0
            if stage >= 4:
                lax.fori_loop(0, L, scale_body, 0)

            pltpu.sync_copy(rows, acc_msg.at[lidx], add=True)
            pltpu.sync_copy(exden, acc_den.at[lidx], add=True)
            return 0
        if stage >= 2:
            lax.fori_loop(0, (cnt + L - 1) // L, grp_body, 0)
        plsc.subcore_barrier()

        wr = pl.multiple_of(sid * (CH // NS), 8)
        pltpu.sync_copy(acc_msg.at[pl.ds(wr, CH // NS)],
                        raw_hbm.at[pl.ds(base + wr, CH // NS)])
        pltpu.sync_copy(acc_den.at[pl.ds(wr, CH // NS)],
                        den_hbm.at[pl.ds(base + wr, CH // NS)])
        plsc.subcore_barrier()
        return 0

    lax.fori_loop(0, CPC, chunk_body, 0)


def _sc_edge(h, ab, auxc, auxm, src, dst, ew, F, H, stage=99):
    mesh = plsc.VectorSubcoreMesh(core_axis_name="c", subcore_axis_name="s")
    kern = pl.kernel(
        functools.partial(_sc_edge_body, F, H, stage),
        out_type=[
            jax.ShapeDtypeStruct((NPAD, F), jnp.float32),
            jax.ShapeDtypeStruct((NPAD, 16), jnp.float32),
        ],
        mesh=mesh,
        compiler_params=pltpu.CompilerParams(
            needs_layout_passes=False, use_tc_tiling_on_sc=False),
        scratch_types=[
            pltpu.VMEM((EPT,), jnp.int32),
            pltpu.VMEM((EPT,), jnp.int32),
            pltpu.VMEM((EPT,), jnp.float32),
            pltpu.VMEM((SEL_CAP,), jnp.int32),
            pltpu.VMEM((L, F), jnp.float32),
            pltpu.VMEM((L, 16), jnp.float32),
            pltpu.VMEM((L, 16), jnp.float32),
            pltpu.VMEM((L, 16), jnp.float32),
            pltpu.VMEM((HEADS, 128), jnp.float32),
            pltpu.VMEM((HEADS, 128), jnp.float32),
            pltpu.VMEM((L,), jnp.int32),
            pltpu.VMEM((L,), jnp.int32),
            pltpu.VMEM((L,), jnp.int32),
            pltpu.VMEM((ZR_PER_TILE, 16), jnp.float32),
            pltpu.MemorySpace.VMEM_SHARED((ACC_ROWS, F), jnp.float32),
            pltpu.MemorySpace.VMEM_SHARED((ACC_ROWS, 16), jnp.float32),
        ],
    )
    zeros8 = jnp.zeros((8, F), jnp.float32)
    return kern(h, ab, auxc, auxm, src, dst, ew, zeros8)


def _blockdiag(att, heads, hid, out_cols):
    a = att.reshape(heads, hid)
    eye = jnp.eye(heads, out_cols, dtype=a.dtype)
    return (a[:, :, None] * eye[:, None, :]).reshape(heads * hid, out_cols)


def kernel(x, edge_index, edge_weight, W1, att_src1, att_dst1, att_edge1, W_edge1, b1,
           W2, att_src2, att_dst2, att_edge2, W_edge2, b2):
    xp = jnp.pad(x, ((0, NPAD - N), (0, 0)))
    src = edge_index[0]
    dst = edge_index[1]

    amat1 = jnp.concatenate([
        _blockdiag(att_src1, HEADS, HID, 8),
        _blockdiag(att_dst1, HEADS, HID, 8)], axis=1)
    we1 = W_edge1.reshape(HEADS, HID)
    ae1 = att_edge1.reshape(HEADS, HID)
    dummy_den = jnp.zeros((NPAD, 16), jnp.float32)
    dummy_b = jnp.zeros((1, IN_CH), jnp.float32)
    dummy_exp = jnp.zeros((HEADS, IN_CH), jnp.float32)

    h1, ab1, auxc1, auxm1 = _mm_att(
        xp, dummy_den, dummy_b, dummy_exp, W1, amat1, we1, ae1, prologue=False)

    raw1, den1 = _sc_edge(h1, ab1, auxc1, auxm1, src, dst, edge_weight,
                          HEADS * HID, HEADS)

    amat2 = jnp.concatenate([
        jnp.pad(_blockdiag(att_src2, 1, OUT_CH, 1), ((0, 0), (0, 7))),
        jnp.pad(_blockdiag(att_dst2, 1, OUT_CH, 1), ((0, 0), (0, 7)))], axis=1)
    we2 = jnp.pad(W_edge2.reshape(1, OUT_CH), ((0, 7), (0, 0)))
    ae2 = jnp.pad(att_edge2.reshape(1, OUT_CH), ((0, 7), (0, 0)))
    expand = _blockdiag(jnp.ones((1, HEADS, HID), jnp.float32), HEADS, HID, 8).T
    b1r = b1.reshape(1, HEADS * HID)

    h2, ab2, auxc2, auxm2 = _mm_att(
        raw1, den1, b1r, expand, W2, amat2, we2, ae2, prologue=True)

    raw2, den2 = _sc_edge(h2, ab2, auxc2, auxm2, src, dst, edge_weight,
                          OUT_CH, 1)

    out = _final(raw2, den2, b2.reshape(1, OUT_CH))
    return out[:N]


# speedup vs baseline: 12.0539x; 1.6887x over previous
import functools

import jax
import jax.numpy as jnp
from jax import lax
from jax.experimental import pallas as pl
from jax.experimental.pallas import tpu as pltpu
from jax.experimental.pallas import tpu_sc as plsc

N = 10000
E = 160000
IN_CH = 256
HID = 256
OUT_CH = 256
HEADS = 8

NPAD = 10240
BLK = 256
NC = 2
NS = 16
L = 16
CH = 256
NCHUNK = NPAD // CH
CPC = NCHUNK // NC
ACC_ROWS = 384
ZR_PER_TILE = ACC_ROWS // NS
EPT = E // NS
SEL_CAP = EPT + L


def _mm_att_body(prologue, raw_ref, den_ref, b_ref, exp_ref,
                 w_ref, amat_ref, we_ref, ae_ref,
                 h_ref, ab_ref, auxc_ref, auxm_ref):
    i = pl.program_id(0)
    nb = pl.num_programs(0)
    if prologue:
        den = den_ref[...][:, :HEADS]
        rden = jnp.where(den > 0, 1.0 / den, 0.0)
        scale = jnp.dot(rden, exp_ref[...], preferred_element_type=jnp.float32)
        hf = raw_ref[...] * scale + b_ref[...]
        x = jnp.where(hf > 0, hf, jnp.exp(jnp.minimum(hf, 0.0)) - 1.0)
    else:
        x = raw_ref[...]
    h = jnp.dot(x, w_ref[...], preferred_element_type=jnp.float32)
    h_ref[...] = h
    ab = jnp.dot(h, amat_ref[...], preferred_element_type=jnp.float32)
    ab_ref[...] = ab
    bmax = jnp.max(ab[:, :HEADS], axis=0)
    colm = jnp.broadcast_to(bmax[:, None], (HEADS, 128))
    c = jnp.sum(we_ref[...] * ae_ref[...], axis=1)

    @pl.when(i == 0)
    def _():
        auxc_ref[...] = jnp.broadcast_to(c[:, None], (HEADS, 128))
        auxm_ref[...] = colm

    @pl.when(i > 0)
    def _():
        auxm_ref[...] = jnp.maximum(auxm_ref[...], colm)

    @pl.when(i == nb - 1)
    def _():
        auxm_ref[...] = auxm_ref[...] + jnp.broadcast_to(
            jnp.maximum(c, 0.0)[:, None], (HEADS, 128))


def _mm_att(x_or_raw, den, b, expand, w, amat, we, ae, prologue):
    n = x_or_raw.shape[0]
    k = x_or_raw.shape[1]
    m = w.shape[1]
    grid = (n // BLK,)
    full = lambda shape: pl.BlockSpec(shape, lambda i: tuple(0 for _ in shape))
    return pl.pallas_call(
        functools.partial(_mm_att_body, prologue),
        grid=grid,
        in_specs=[
            pl.BlockSpec((BLK, k), lambda i: (i, 0)),
            pl.BlockSpec((BLK, 16), lambda i: (i, 0)),
            full((1, k)),
            full((HEADS, k)),
            full((k, m)),
            full((m, 16)),
            full((HEADS, 256)),
            full((HEADS, 256)),
        ],
        out_specs=[
            pl.BlockSpec((BLK, m), lambda i: (i, 0)),
            pl.BlockSpec((BLK, 16), lambda i: (i, 0)),
            full((HEADS, 128)),
            full((HEADS, 128)),
        ],
        out_shape=[
            jax.ShapeDtypeStruct((n, m), jnp.float32),
            jax.ShapeDtypeStruct((n, 16), jnp.float32),
            jax.ShapeDtypeStruct((HEADS, 128), jnp.float32),
            jax.ShapeDtypeStruct((HEADS, 128), jnp.float32),
        ],
    )(x_or_raw, den, b, expand, w, amat, we, ae)


def _final_body(raw_ref, den_ref, b_ref, out_ref):
    den0 = den_ref[...][:, 0:1]
    rden = jnp.where(den0 > 0, 1.0 / den0, 0.0)
    o = raw_ref[...] * rden + b_ref[...]
    t = o - jnp.max(o, axis=1, keepdims=True)
    lse = jnp.log(jnp.sum(jnp.exp(t), axis=1, keepdims=True))
    out_ref[...] = t - lse


def _final(raw, den, b):
    n = raw.shape[0]
    f = raw.shape[1]
    return pl.pallas_call(
        _final_body,
        grid=(n // BLK,),
        in_specs=[
            pl.BlockSpec((BLK, f), lambda i: (i, 0)),
            pl.BlockSpec((BLK, 16), lambda i: (i, 0)),
            pl.BlockSpec((1, f), lambda i: (0, 0)),
        ],
        out_specs=pl.BlockSpec((BLK, f), lambda i: (i, 0)),
        out_shape=jax.ShapeDtypeStruct((n, f), jnp.float32),
    )(raw, den, b)


NBUF = 3


def _sc_edge_body(F, H, G, CH, ACC_ROWS, CPC, ZR,
                  h_hbm, ab_hbm, auxc_hbm, auxm_hbm, src_hbm, dst_hbm,
                  ew_hbm, zeros_hbm, raw_hbm, den_hbm,
                  src_sl, dst_sl, ew_sl, sel, rows, astat, dstat, exden,
                  ewv, srcidx, dstidx, lidx, auxc_v, auxm_v, zden,
                  acc_msg, acc_den, gsem, ssem):
    cid = lax.axis_index("c")
    sid = lax.axis_index("s")
    iot = lax.iota(jnp.int32, L)
    zeros16 = jnp.zeros((L,), jnp.float32)
    laneG = iot < G

    ebase = sid * EPT
    pltpu.sync_copy(src_hbm.at[pl.ds(ebase, EPT)], src_sl)
    pltpu.sync_copy(dst_hbm.at[pl.ds(ebase, EPT)], dst_sl)
    pltpu.sync_copy(ew_hbm.at[pl.ds(ebase, EPT)], ew_sl)
    pltpu.sync_copy(auxc_hbm, auxc_v)
    pltpu.sync_copy(auxm_hbm, auxm_v)

    def zdloop(j, _):
        zden[j, pl.ds(0, L)] = zeros16
        return 0
    lax.fori_loop(0, ZR, zdloop, 0)

    for b in range(NBUF):
        for h in range(H, 16):
            plsc.store_scatter(exden[b], [iot, jnp.full((L,), h, jnp.int32)],
                               zeros16, mask=laneG)

    def chunk_body(ci, _):
        base = pl.multiple_of((ci * NC + cid) * CH, 8)
        zr = pl.multiple_of(sid * ZR, 8)
        for j in range(ZR // 8):
            pltpu.sync_copy(zeros_hbm, acc_msg.at[pl.ds(zr + j * 8, 8)])
        pltpu.sync_copy(zden, acc_den.at[pl.ds(zr, ZR)])
        plsc.subcore_barrier()

        def scan_body(j, cnt):
            d16 = dst_sl[pl.ds(j * L, L)]
            msk = (d16 >= base) & (d16 < base + CH)
            pref = plsc.cumsum(msk.astype(jnp.int32))
            plsc.store_scatter(sel, [cnt + pref - 1], j * L + iot, mask=msk)
            return cnt + pref[L - 1]
        cnt = lax.fori_loop(0, EPT // L, scan_body, jnp.int32(0))
        ngrp = (cnt + G - 1) // G

        def prep(g, b):
            ids = sel[pl.ds(g * G, L)]
            valid = ((g * G + iot) < cnt) & laneG
            ids = jnp.where(valid, ids, 0)
            s16 = plsc.load_gather(src_sl, [ids])
            d16 = plsc.load_gather(dst_sl, [ids])
            e16 = plsc.load_gather(ew_sl, [ids])
            plsc.store_scatter(srcidx[b], [iot], s16, mask=laneG)
            plsc.store_scatter(dstidx[b], [iot], d16, mask=laneG)
            plsc.store_scatter(lidx[b], [iot],
                               jnp.where(valid, d16 - base, CH + 8 * sid),
                               mask=laneG)
            plsc.store_scatter(ewv[b], [iot], e16, mask=laneG)

        def issue_gathers(b):
            pltpu.async_copy(h_hbm.at[srcidx[b]], rows[b], gsem[b])
            pltpu.async_copy(ab_hbm.at[srcidx[b]], astat[b], gsem[b])
            pltpu.async_copy(ab_hbm.at[dstidx[b]], dstat[b], gsem[b])

        def wait_gathers(b):
            pltpu.make_async_copy(h_hbm.at[srcidx[b]], rows[b], gsem[b]).wait()
            pltpu.make_async_copy(ab_hbm.at[srcidx[b]], astat[b], gsem[b]).wait()
            pltpu.make_async_copy(ab_hbm.at[dstidx[b]], dstat[b], gsem[b]).wait()

        def issue_scatter(b):
            pltpu.async_copy(rows[b], acc_msg.at[lidx[b]], ssem[b], add=True)
            pltpu.async_copy(exden[b], acc_den.at[lidx[b]], ssem[b], add=True)

        def wait_scatter(b):
            pltpu.make_async_copy(rows[b], acc_msg.at[lidx[b]], ssem[b]).wait()
            pltpu.make_async_copy(exden[b], acc_den.at[lidx[b]], ssem[b]).wait()

        def compute(g, b):
            valid = ((g * G + iot) < cnt) & laneG
            ew16 = ewv[b][pl.ds(0, G)] if G == L else None
            if ew16 is None:
                ew16 = plsc.load_gather(ewv[b], [jnp.where(laneG, iot, 0)])
            for h in range(H):
                hcol = jnp.full((L,), h, jnp.int32)
                a_s = plsc.load_gather(astat[b], [jnp.where(laneG, iot, 0), hcol])
                a_d = plsc.load_gather(dstat[b], [jnp.where(laneG, iot, 0), hcol + 8])
                c_h = auxc_v[h, pl.ds(0, L)][0]
                mp_h = auxm_v[h, pl.ds(0, L)][0]
                al = a_s + a_d + ew16 * c_h
                al = jnp.where(al > 0, al, 0.2 * al)
                mb = a_d + mp_h
                mb = jnp.where(mb > 0, mb, 0.2 * mb)
                exh = jnp.where(valid, jnp.exp(al - mb), 0.0)
                plsc.store_scatter(exden[b], [iot, hcol], exh, mask=laneG)

            def scale_body(i, _):
                exr = exden[b][i, pl.ds(0, L)]
                for h in range(H):
                    e_s = exr[h]
                    for k in range(256 // L):
                        sl = pl.ds(h * 256 + k * L, L)
                        rows[b][i, sl] = rows[b][i, sl] * e_s
                return 0
            lax.fori_loop(0, G, scale_body, 0)

        @pl.when(ngrp > 0)
        def _():
            prep(jnp.int32(0), 0)
            issue_gathers(0)

        def tri_body(t, _):
            for j in range(NBUF):
                g = t * NBUF + j

                @pl.when(g < ngrp)
                def _():
                    nb = (j + 1) % NBUF

                    @pl.when(g + 1 < ngrp)
                    def _():
                        @pl.when(g >= NBUF - 1)
                        def _():
                            wait_scatter(nb)
                        prep(g + 1, nb)
                        issue_gathers(nb)

                    wait_gathers(j)
                    compute(g, j)
                    issue_scatter(j)
            return 0
        lax.fori_loop(0, (ngrp + NBUF - 1) // NBUF, tri_body, 0)

        for j in range(NBUF):
            conds = []
            for k in range(NBUF):
                conds.append((ngrp >= k + 1) & ((ngrp - 1 - k) % NBUF == j))
            cond = conds[0] | conds[1] | conds[2]

            @pl.when(cond)
            def _():
                wait_scatter(j)
        plsc.subcore_barrier()

        wr = pl.multiple_of(sid * (CH // NS), 8)
        pltpu.sync_copy(acc_msg.at[pl.ds(wr, CH // NS)],
                        raw_hbm.at[pl.ds(base + wr, CH // NS)])
        pltpu.sync_copy(acc_den.at[pl.ds(wr, CH // NS)],
                        den_hbm.at[pl.ds(base + wr, CH // NS)])
        plsc.subcore_barrier()
        return 0

    lax.fori_loop(0, CPC, chunk_body, 0)


def _sc_edge(h, ab, auxc, auxm, src, dst, ew, F, H):
    G = 8 if F == 2048 else 16
    CH = 128 if F == 2048 else 1024
    ACC_ROWS = CH + 128
    CPC = NPAD // CH // NC
    ZR = ACC_ROWS // NS
    mesh = plsc.VectorSubcoreMesh(core_axis_name="c", subcore_axis_name="s")
    kern = pl.kernel(
        functools.partial(_sc_edge_body, F, H, G, CH, ACC_ROWS, CPC, ZR),
        out_type=[
            jax.ShapeDtypeStruct((NPAD, F), jnp.float32),
            jax.ShapeDtypeStruct((NPAD, 16), jnp.float32),
        ],
        mesh=mesh,
        compiler_params=pltpu.CompilerParams(
            needs_layout_passes=False, use_tc_tiling_on_sc=False),
        scratch_types=[
            pltpu.VMEM((EPT,), jnp.int32),
            pltpu.VMEM((EPT,), jnp.int32),
            pltpu.VMEM((EPT,), jnp.float32),
            pltpu.VMEM((SEL_CAP,), jnp.int32),
            [pltpu.VMEM((G, F), jnp.float32) for _ in range(NBUF)],
            [pltpu.VMEM((G, 16), jnp.float32) for _ in range(NBUF)],
            [pltpu.VMEM((G, 16), jnp.float32) for _ in range(NBUF)],
            [pltpu.VMEM((G, 16), jnp.float32) for _ in range(NBUF)],
            [pltpu.VMEM((G,), jnp.float32) for _ in range(NBUF)],
            [pltpu.VMEM((G,), jnp.int32) for _ in range(NBUF)],
            [pltpu.VMEM((G,), jnp.int32) for _ in range(NBUF)],
            [pltpu.VMEM((G,), jnp.int32) for _ in range(NBUF)],
            pltpu.VMEM((HEADS, 128), jnp.float32),
            pltpu.VMEM((HEADS, 128), jnp.float32),
            pltpu.VMEM(((CH + 128) // NS, 16), jnp.float32),
            pltpu.MemorySpace.VMEM_SHARED((ACC_ROWS, F), jnp.float32),
            pltpu.MemorySpace.VMEM_SHARED((ACC_ROWS, 16), jnp.float32),
            [pltpu.SemaphoreType.DMA for _ in range(NBUF)],
            [pltpu.SemaphoreType.DMA for _ in range(NBUF)],
        ],
    )
    zeros8 = jnp.zeros((8, F), jnp.float32)
    return kern(h, ab, auxc, auxm, src, dst, ew, zeros8)


def _blockdiag(att, heads, hid, out_cols):
    a = att.reshape(heads, hid)
    eye = jnp.eye(heads, out_cols, dtype=a.dtype)
    return (a[:, :, None] * eye[:, None, :]).reshape(heads * hid, out_cols)


def kernel(x, edge_index, edge_weight, W1, att_src1, att_dst1, att_edge1, W_edge1, b1,
           W2, att_src2, att_dst2, att_edge2, W_edge2, b2):
    xp = jnp.pad(x, ((0, NPAD - N), (0, 0)))
    src = edge_index[0]
    dst = edge_index[1]

    amat1 = jnp.concatenate([
        _blockdiag(att_src1, HEADS, HID, 8),
        _blockdiag(att_dst1, HEADS, HID, 8)], axis=1)
    we1 = W_edge1.reshape(HEADS, HID)
    ae1 = att_edge1.reshape(HEADS, HID)
    dummy_den = jnp.zeros((NPAD, 16), jnp.float32)
    dummy_b = jnp.zeros((1, IN_CH), jnp.float32)
    dummy_exp = jnp.zeros((HEADS, IN_CH), jnp.float32)

    h1, ab1, auxc1, auxm1 = _mm_att(
        xp, dummy_den, dummy_b, dummy_exp, W1, amat1, we1, ae1, prologue=False)

    raw1, den1 = _sc_edge(h1, ab1, auxc1, auxm1, src, dst, edge_weight,
                          HEADS * HID, HEADS)

    amat2 = jnp.concatenate([
        jnp.pad(_blockdiag(att_src2, 1, OUT_CH, 1), ((0, 0), (0, 7))),
        jnp.pad(_blockdiag(att_dst2, 1, OUT_CH, 1), ((0, 0), (0, 7)))], axis=1)
    we2 = jnp.pad(W_edge2.reshape(1, OUT_CH), ((0, 7), (0, 0)))
    ae2 = jnp.pad(att_edge2.reshape(1, OUT_CH), ((0, 7), (0, 0)))
    expand = _blockdiag(jnp.ones((1, HEADS, HID), jnp.float32), HEADS, HID, 8).T
    b1r = b1.reshape(1, HEADS * HID)

    h2, ab2, auxc2, auxm2 = _mm_att(
        raw1, den1, b1r, expand, W2, amat2, we2, ae2, prologue=True)

    raw2, den2 = _sc_edge(h2, ab2, auxc2, auxm2, src, dst, edge_weight,
                          OUT_CH, 1)

    out = _final(raw2, den2, b2.reshape(1, OUT_CH))
    return out[:N]


# speedup vs baseline: 12.5825x; 1.0439x over previous
import functools

import jax
import jax.numpy as jnp
from jax import lax
from jax.experimental import pallas as pl
from jax.experimental.pallas import tpu as pltpu
from jax.experimental.pallas import tpu_sc as plsc

N = 10000
E = 160000
IN_CH = 256
HID = 256
OUT_CH = 256
HEADS = 8

NPAD = 10240
BLK = 256
NC = 2
NS = 16
L = 16
CH = 256
NCHUNK = NPAD // CH
CPC = NCHUNK // NC
ACC_ROWS = 384
ZR_PER_TILE = ACC_ROWS // NS
EPT = E // NS
SEL_CAP = EPT + L


def _mm_att_body(prologue, raw_ref, den_ref, b_ref, exp_ref,
                 w_ref, amat_ref, we_ref, ae_ref,
                 h_ref, ab_ref, auxc_ref, auxm_ref):
    i = pl.program_id(0)
    nb = pl.num_programs(0)
    if prologue:
        den = den_ref[...][:, :HEADS]
        rden = jnp.where(den > 0, 1.0 / den, 0.0)
        scale = jnp.dot(rden, exp_ref[...], preferred_element_type=jnp.float32)
        hf = raw_ref[...] * scale + b_ref[...]
        x = jnp.where(hf > 0, hf, jnp.exp(jnp.minimum(hf, 0.0)) - 1.0)
    else:
        x = raw_ref[...]
    h = jnp.dot(x, w_ref[...], preferred_element_type=jnp.float32)
    h_ref[...] = h
    ab = jnp.dot(h, amat_ref[...], preferred_element_type=jnp.float32)
    ab_ref[...] = ab
    bmax = jnp.max(ab[:, :HEADS], axis=0)
    colm = jnp.broadcast_to(bmax[:, None], (HEADS, 128))
    c = jnp.sum(we_ref[...] * ae_ref[...], axis=1)

    @pl.when(i == 0)
    def _():
        auxc_ref[...] = jnp.broadcast_to(c[:, None], (HEADS, 128))
        auxm_ref[...] = colm

    @pl.when(i > 0)
    def _():
        auxm_ref[...] = jnp.maximum(auxm_ref[...], colm)

    @pl.when(i == nb - 1)
    def _():
        auxm_ref[...] = auxm_ref[...] + jnp.broadcast_to(
            jnp.maximum(c, 0.0)[:, None], (HEADS, 128))


def _mm_att(x_or_raw, den, b, expand, w, amat, we, ae, prologue):
    n = x_or_raw.shape[0]
    k = x_or_raw.shape[1]
    m = w.shape[1]
    grid = (n // BLK,)
    full = lambda shape: pl.BlockSpec(shape, lambda i: tuple(0 for _ in shape))
    return pl.pallas_call(
        functools.partial(_mm_att_body, prologue),
        grid=grid,
        in_specs=[
            pl.BlockSpec((BLK, k), lambda i: (i, 0)),
            pl.BlockSpec((BLK, 16), lambda i: (i, 0)),
            full((1, k)),
            full((HEADS, k)),
            full((k, m)),
            full((m, 16)),
            full((HEADS, 256)),
            full((HEADS, 256)),
        ],
        out_specs=[
            pl.BlockSpec((BLK, m), lambda i: (i, 0)),
            pl.BlockSpec((BLK, 16), lambda i: (i, 0)),
            full((HEADS, 128)),
            full((HEADS, 128)),
        ],
        out_shape=[
            jax.ShapeDtypeStruct((n, m), jnp.float32),
            jax.ShapeDtypeStruct((n, 16), jnp.float32),
            jax.ShapeDtypeStruct((HEADS, 128), jnp.float32),
            jax.ShapeDtypeStruct((HEADS, 128), jnp.float32),
        ],
    )(x_or_raw, den, b, expand, w, amat, we, ae)


def _final_body(raw_ref, den_ref, b_ref, out_ref):
    den0 = den_ref[...][:, 0:1]
    rden = jnp.where(den0 > 0, 1.0 / den0, 0.0)
    o = raw_ref[...] * rden + b_ref[...]
    t = o - jnp.max(o, axis=1, keepdims=True)
    lse = jnp.log(jnp.sum(jnp.exp(t), axis=1, keepdims=True))
    out_ref[...] = t - lse


def _final(raw, den, b):
    n = raw.shape[0]
    f = raw.shape[1]
    return pl.pallas_call(
        _final_body,
        grid=(n // BLK,),
        in_specs=[
            pl.BlockSpec((BLK, f), lambda i: (i, 0)),
            pl.BlockSpec((BLK, 16), lambda i: (i, 0)),
            pl.BlockSpec((1, f), lambda i: (0, 0)),
        ],
        out_specs=pl.BlockSpec((BLK, f), lambda i: (i, 0)),
        out_shape=jax.ShapeDtypeStruct((n, f), jnp.float32),
    )(raw, den, b)


NBUF = 3


def _sc_edge_body(F, H, G, CH, ACC_ROWS, CPC, ZR,
                  h_hbm, ab_hbm, auxc_hbm, auxm_hbm, src_hbm, dst_hbm,
                  ew_hbm, zeros_hbm, raw_hbm, den_hbm,
                  src_sl, dst_sl, ew_sl, sel, rows, astat, dstat, exden,
                  ewv, srcidx, dstidx, lidx, auxc_v, auxm_v, zden,
                  acc_msg, acc_den, gsem, ssem):
    cid = lax.axis_index("c")
    sid = lax.axis_index("s")
    iot = lax.iota(jnp.int32, L)
    zeros16 = jnp.zeros((L,), jnp.float32)
    laneG = iot < G

    ebase = sid * EPT
    pltpu.sync_copy(src_hbm.at[pl.ds(ebase, EPT)], src_sl)
    pltpu.sync_copy(dst_hbm.at[pl.ds(ebase, EPT)], dst_sl)
    pltpu.sync_copy(ew_hbm.at[pl.ds(ebase, EPT)], ew_sl)
    pltpu.sync_copy(auxc_hbm, auxc_v)
    pltpu.sync_copy(auxm_hbm, auxm_v)

    def zdloop(j, _):
        zden[j, pl.ds(0, L)] = zeros16
        return 0
    lax.fori_loop(0, ZR, zdloop, 0)

    for b in range(NBUF):
        for h in range(H, 16):
            plsc.store_scatter(exden[b], [iot, jnp.full((L,), h, jnp.int32)],
                               zeros16, mask=laneG)

    def chunk_body(ci, _):
        base = pl.multiple_of((ci * NC + cid) * CH, 8)
        zr = pl.multiple_of(sid * ZR, 8)
        for j in range(ZR // 8):
            pltpu.sync_copy(zeros_hbm, acc_msg.at[pl.ds(zr + j * 8, 8)])
        pltpu.sync_copy(zden, acc_den.at[pl.ds(zr, ZR)])
        plsc.subcore_barrier()

        def scan_body(j, cnt):
            d16 = dst_sl[pl.ds(j * L, L)]
            msk = (d16 >= base) & (d16 < base + CH)
            pref = plsc.cumsum(msk.astype(jnp.int32))
            plsc.store_scatter(sel, [cnt + pref - 1], j * L + iot, mask=msk)
            return cnt + pref[L - 1]
        cnt = lax.fori_loop(0, EPT // L, scan_body, jnp.int32(0))
        ngrp = (cnt + G - 1) // G

        def prep(g, b):
            ids = sel[pl.ds(g * G, L)]
            valid = ((g * G + iot) < cnt) & laneG
            ids = jnp.where(valid, ids, 0)
            s16 = plsc.load_gather(src_sl, [ids])
            d16 = plsc.load_gather(dst_sl, [ids])
            e16 = plsc.load_gather(ew_sl, [ids])
            plsc.store_scatter(srcidx[b], [iot], s16, mask=laneG)
            plsc.store_scatter(dstidx[b], [iot], d16, mask=laneG)
            plsc.store_scatter(lidx[b], [iot],
                               jnp.where(valid, d16 - base, CH + 8 * sid),
                               mask=laneG)
            plsc.store_scatter(ewv[b], [iot], e16, mask=laneG)

        def issue_gathers(b):
            pltpu.async_copy(h_hbm.at[srcidx[b]], rows[b], gsem[b])
            pltpu.async_copy(ab_hbm.at[srcidx[b]], astat[b], gsem[b])
            pltpu.async_copy(ab_hbm.at[dstidx[b]], dstat[b], gsem[b])

        def wait_gathers(b):
            pltpu.make_async_copy(h_hbm.at[srcidx[b]], rows[b], gsem[b]).wait()
            pltpu.make_async_copy(ab_hbm.at[srcidx[b]], astat[b], gsem[b]).wait()
            pltpu.make_async_copy(ab_hbm.at[dstidx[b]], dstat[b], gsem[b]).wait()

        def issue_scatter(b):
            pltpu.async_copy(rows[b], acc_msg.at[lidx[b]], ssem[b], add=True)
            pltpu.async_copy(exden[b], acc_den.at[lidx[b]], ssem[b], add=True)

        def wait_scatter(b):
            pltpu.make_async_copy(rows[b], acc_msg.at[lidx[b]], ssem[b]).wait()
            pltpu.make_async_copy(exden[b], acc_den.at[lidx[b]], ssem[b]).wait()

        def compute(g, b):
            valid = ((g * G + iot) < cnt) & laneG
            ew16 = ewv[b][pl.ds(0, G)] if G == L else None
            if ew16 is None:
                ew16 = plsc.load_gather(ewv[b], [jnp.where(laneG, iot, 0)])
            for h in range(H):
                hcol = jnp.full((L,), h, jnp.int32)
                a_s = plsc.load_gather(astat[b], [jnp.where(laneG, iot, 0), hcol])
                a_d = plsc.load_gather(dstat[b], [jnp.where(laneG, iot, 0), hcol + 8])
                c_h = auxc_v[h, pl.ds(0, L)][0]
                mp_h = auxm_v[h, pl.ds(0, L)][0]
                al = a_s + a_d + ew16 * c_h
                al = jnp.where(al > 0, al, 0.2 * al)
                mb = a_d + mp_h
                mb = jnp.where(mb > 0, mb, 0.2 * mb)
                exh = jnp.where(valid, jnp.exp(al - mb), 0.0)
                plsc.store_scatter(exden[b], [iot, hcol], exh, mask=laneG)

            def scale_body(i, _):
                exr = exden[b][i, pl.ds(0, L)]
                for h in range(H):
                    e_s = exr[h]
                    for k in range(256 // L):
                        sl = pl.ds(h * 256 + k * L, L)
                        rows[b][i, sl] = rows[b][i, sl] * e_s
                return 0
            lax.fori_loop(0, G, scale_body, 0)

        @pl.when(ngrp > 0)
        def _():
            prep(jnp.int32(0), 0)
            issue_gathers(0)

        def tri_body(t, _):
            for j in range(NBUF):
                g = t * NBUF + j

                @pl.when(g < ngrp)
                def _():
                    nb = (j + 1) % NBUF

                    @pl.when(g + 1 < ngrp)
                    def _():
                        @pl.when(g >= NBUF - 1)
                        def _():
                            wait_scatter(nb)
                        prep(g + 1, nb)
                        issue_gathers(nb)

                    wait_gathers(j)
                    compute(g, j)
                    issue_scatter(j)
            return 0
        lax.fori_loop(0, (ngrp + NBUF - 1) // NBUF, tri_body, 0)

        for j in range(NBUF):
            conds = []
            for k in range(NBUF):
                conds.append((ngrp >= k + 1) & ((ngrp - 1 - k) % NBUF == j))
            cond = conds[0] | conds[1] | conds[2]

            @pl.when(cond)
            def _():
                wait_scatter(j)
        plsc.subcore_barrier()

        wr = pl.multiple_of(sid * (CH // NS), 8)
        pltpu.sync_copy(acc_msg.at[pl.ds(wr, CH // NS)],
                        raw_hbm.at[pl.ds(base + wr, CH // NS)])
        pltpu.sync_copy(acc_den.at[pl.ds(wr, CH // NS)],
                        den_hbm.at[pl.ds(base + wr, CH // NS)])
        plsc.subcore_barrier()
        return 0

    lax.fori_loop(0, CPC, chunk_body, 0)


def _sc_edge(h, ab, auxc, auxm, src, dst, ew, F, H):
    G = 8 if F == 2048 else 16
    CH = 128 if F == 2048 else 1024
    ACC_ROWS = CH + 128
    CPC = NPAD // CH // NC
    ZR = CH // NS
    mesh = plsc.VectorSubcoreMesh(core_axis_name="c", subcore_axis_name="s")
    kern = pl.kernel(
        functools.partial(_sc_edge_body, F, H, G, CH, ACC_ROWS, CPC, ZR),
        out_type=[
            jax.ShapeDtypeStruct((NPAD, F), jnp.float32),
            jax.ShapeDtypeStruct((NPAD, 16), jnp.float32),
        ],
        mesh=mesh,
        compiler_params=pltpu.CompilerParams(
            needs_layout_passes=False, use_tc_tiling_on_sc=False),
        scratch_types=[
            pltpu.VMEM((EPT,), jnp.int32),
            pltpu.VMEM((EPT,), jnp.int32),
            pltpu.VMEM((EPT,), jnp.float32),
            pltpu.VMEM((SEL_CAP,), jnp.int32),
            [pltpu.VMEM((G, F), jnp.float32) for _ in range(NBUF)],
            [pltpu.VMEM((G, 16), jnp.float32) for _ in range(NBUF)],
            [pltpu.VMEM((G, 16), jnp.float32) for _ in range(NBUF)],
            [pltpu.VMEM((G, 16), jnp.float32) for _ in range(NBUF)],
            [pltpu.VMEM((G,), jnp.float32) for _ in range(NBUF)],
            [pltpu.VMEM((G,), jnp.int32) for _ in range(NBUF)],
            [pltpu.VMEM((G,), jnp.int32) for _ in range(NBUF)],
            [pltpu.VMEM((G,), jnp.int32) for _ in range(NBUF)],
            pltpu.VMEM((HEADS, 128), jnp.float32),
            pltpu.VMEM((HEADS, 128), jnp.float32),
            pltpu.VMEM((CH // NS, 16), jnp.float32),
            pltpu.MemorySpace.VMEM_SHARED((ACC_ROWS, F), jnp.float32),
            pltpu.MemorySpace.VMEM_SHARED((ACC_ROWS, 16), jnp.float32),
            [pltpu.SemaphoreType.DMA for _ in range(NBUF)],
            [pltpu.SemaphoreType.DMA for _ in range(NBUF)],
        ],
    )
    zeros8 = jnp.zeros((8, F), jnp.float32)
    return kern(h, ab, auxc, auxm, src, dst, ew, zeros8)


def _blockdiag(att, heads, hid, out_cols):
    a = att.reshape(heads, hid)
    eye = jnp.eye(heads, out_cols, dtype=a.dtype)
    return (a[:, :, None] * eye[:, None, :]).reshape(heads * hid, out_cols)


def kernel(x, edge_index, edge_weight, W1, att_src1, att_dst1, att_edge1, W_edge1, b1,
           W2, att_src2, att_dst2, att_edge2, W_edge2, b2):
    xp = jnp.pad(x, ((0, NPAD - N), (0, 0)))
    src = edge_index[0]
    dst = edge_index[1]

    amat1 = jnp.concatenate([
        _blockdiag(att_src1, HEADS, HID, 8),
        _blockdiag(att_dst1, HEADS, HID, 8)], axis=1)
    we1 = W_edge1.reshape(HEADS, HID)
    ae1 = att_edge1.reshape(HEADS, HID)
    dummy_den = jnp.zeros((NPAD, 16), jnp.float32)
    dummy_b = jnp.zeros((1, IN_CH), jnp.float32)
    dummy_exp = jnp.zeros((HEADS, IN_CH), jnp.float32)

    h1, ab1, auxc1, auxm1 = _mm_att(
        xp, dummy_den, dummy_b, dummy_exp, W1, amat1, we1, ae1, prologue=False)

    raw1, den1 = _sc_edge(h1, ab1, auxc1, auxm1, src, dst, edge_weight,
                          HEADS * HID, HEADS)

    amat2 = jnp.concatenate([
        jnp.pad(_blockdiag(att_src2, 1, OUT_CH, 1), ((0, 0), (0, 7))),
        jnp.pad(_blockdiag(att_dst2, 1, OUT_CH, 1), ((0, 0), (0, 7)))], axis=1)
    we2 = jnp.pad(W_edge2.reshape(1, OUT_CH), ((0, 7), (0, 0)))
    ae2 = jnp.pad(att_edge2.reshape(1, OUT_CH), ((0, 7), (0, 0)))
    expand = _blockdiag(jnp.ones((1, HEADS, HID), jnp.float32), HEADS, HID, 8).T
    b1r = b1.reshape(1, HEADS * HID)

    h2, ab2, auxc2, auxm2 = _mm_att(
        raw1, den1, b1r, expand, W2, amat2, we2, ae2, prologue=True)

    raw2, den2 = _sc_edge(h2, ab2, auxc2, auxm2, src, dst, edge_weight,
                          OUT_CH, 1)

    out = _final(raw2, den2, b2.reshape(1, OUT_CH))
    return out[:N]
